# baseline (device time: 1264052 ns/iter reference)
import jax
import jax.numpy as jnp
from jax import lax
from jax.experimental import pallas as pl
from jax.experimental.pallas import tpu as pltpu

N_DEV = 16
E_LOCAL = 4
N_EXP = 64
D = 512
H = 1024
T = 1024


def kernel(x, router_W, route_idx, expert_W):
    scores = jnp.dot(x, router_W)
    probs = jax.nn.softmax(scores, axis=-1)
    oh = (jax.nn.one_hot(route_idx[:, 0], N_EXP, dtype=x.dtype)
          + jax.nn.one_hot(route_idx[:, 1], N_EXP, dtype=x.dtype))
    wg = probs * oh
    wg = wg / jnp.sum(wg, axis=-1, keepdims=True)
    my = lax.axis_index("i")
    wg_rot = jnp.roll(wg, shift=-E_LOCAL * my, axis=1)
    xw = jnp.concatenate([x, wg_rot], axis=1)
    W = expert_W.reshape(E_LOCAL * D, H)

    def body(xw_ref, W_ref, out_ref, xw_buf, acc_buf,
             xw_send, xw_recv, acc_send, acc_recv,
             out_send, out_recv, credit_sem):
        my_pos = lax.axis_index("i")
        left = lax.rem(my_pos - 1 + N_DEV, N_DEV)
        right = lax.rem(my_pos + 1, N_DEV)

        barrier = pltpu.get_barrier_semaphore()
        for nbr in (left, right):
            pl.semaphore_signal(barrier, inc=1, device_id=(nbr,),
                                device_id_type=pl.DeviceIdType.MESH)
        pl.semaphore_wait(barrier, 2)

        def add_contrib(h, slot, init):
            xc = xw_buf[slot, :, 0:D]
            total = None
            for j in range(E_LOCAL):
                col = D + E_LOCAL * h + j
                g = xw_buf[slot, :, col:col + 1]
                p = jnp.dot(xc * g, W_ref[D * j:D * (j + 1), :],
                            preferred_element_type=jnp.float32)
                total = p if total is None else total + p
            if init:
                acc_buf[slot] = total
            else:
                acc_buf[slot] = acc_buf[slot] + total

        xw_buf[0] = xw_ref[...]
        add_contrib(0, 0, init=True)

        for h in range(1, N_DEV):
            send_slot = (h - 1) % 2
            recv_slot = h % 2
            if h >= 2:
                pl.semaphore_wait(credit_sem, 1)
            rd_xw = pltpu.make_async_remote_copy(
                src_ref=xw_buf.at[send_slot],
                dst_ref=xw_buf.at[recv_slot],
                send_sem=xw_send.at[send_slot],
                recv_sem=xw_recv.at[recv_slot],
                device_id=(right,),
                device_id_type=pl.DeviceIdType.MESH,
            )
            rd_acc = pltpu.make_async_remote_copy(
                src_ref=acc_buf.at[send_slot],
                dst_ref=acc_buf.at[recv_slot],
                send_sem=acc_send.at[send_slot],
                recv_sem=acc_recv.at[recv_slot],
                device_id=(right,),
                device_id_type=pl.DeviceIdType.MESH,
            )
            rd_xw.start()
            rd_acc.start()
            rd_xw.wait()
            rd_acc.wait()
            if h <= N_DEV - 2:
                pl.semaphore_signal(credit_sem, inc=1, device_id=(left,),
                                    device_id_type=pl.DeviceIdType.MESH)
            add_contrib(h, recv_slot, init=False)

        rd_out = pltpu.make_async_remote_copy(
            src_ref=acc_buf.at[(N_DEV - 1) % 2],
            dst_ref=out_ref,
            send_sem=out_send,
            recv_sem=out_recv,
            device_id=(right,),
            device_id_type=pl.DeviceIdType.MESH,
        )
        rd_out.start()
        rd_out.wait()

    return pl.pallas_call(
        body,
        out_shape=jax.ShapeDtypeStruct((T, H), jnp.float32),
        in_specs=[
            pl.BlockSpec(memory_space=pltpu.VMEM),
            pl.BlockSpec(memory_space=pltpu.VMEM),
        ],
        out_specs=pl.BlockSpec(memory_space=pltpu.VMEM),
        scratch_shapes=[
            pltpu.VMEM((2, T, D + N_EXP), jnp.float32),
            pltpu.VMEM((2, T, H), jnp.float32),
            pltpu.SemaphoreType.DMA((2,)),
            pltpu.SemaphoreType.DMA((2,)),
            pltpu.SemaphoreType.DMA((2,)),
            pltpu.SemaphoreType.DMA((2,)),
            pltpu.SemaphoreType.DMA,
            pltpu.SemaphoreType.DMA,
            pltpu.SemaphoreType.REGULAR,
        ],
        compiler_params=pltpu.CompilerParams(collective_id=0),
    )(xw, W)
